# baseline (speedup 1.0000x reference)
import functools

import jax
import jax.numpy as jnp
import numpy as np
from jax import lax
from jax.experimental import pallas as pl
from jax.experimental.pallas import tpu as pltpu
from jax.experimental.pallas import tpu_sc as plsc

B = 1024
D = 128
V = 100000
K = 101
ROWS = 2 * B

NC, NS, L = 2, 16, 16
NW = NC * NS
RPW = ROWS // NW
NVREG = V // L
NBLK = NVREG // 2
CAP = 2048
T0 = 0.9975
T0_BITS = int(np.float32(T0).view(np.int32))
HI_BITS = 0x7F800000


def _hs(t):
    return jnp.clip((t + 3.0) / 6.0, 0.0, 1.0)


def _splat(x, dtype=None):
    v = jnp.broadcast_to(x, (L,))
    return v.astype(dtype) if dtype is not None else v


def _sc_stats_body(cos_hbm, s_hbm, ss_hbm, t_hbm, buf, cand, offref, s_loc,
                   ss_loc, t_loc):
    wid = lax.axis_index("s") * NC + lax.axis_index("c")
    row0 = wid * RPW
    zeros16i = jnp.zeros((L,), jnp.int32)
    zeros16f = jnp.zeros((L,), jnp.float32)
    t0v = jnp.full((L,), T0, jnp.float32)
    lanes = lax.broadcasted_iota(jnp.int32, (L,), 0)

    def row_body(r_local, carry):
        sacc, ssacc, tacc = carry
        row = row0 + r_local
        pltpu.sync_copy(cos_hbm.at[row], buf)
        offref[...] = zeros16i

        def block(i, unused):
            v0 = buf[pl.ds(i * (2 * L), L)]
            v1 = buf[pl.ds(i * (2 * L) + L, L)]
            hit = jnp.max(jnp.maximum(v0, v1)) > T0

            @pl.when(hit)
            def _commit():
                offv = offref[...]
                o = offv[0]
                c0 = jnp.sum((v0 > t0v).astype(jnp.int32))
                c1 = jnp.sum((v1 > t0v).astype(jnp.int32))
                k0, _ = plsc.sort_key_val(v0, v0, descending=True)
                k1, _ = plsc.sort_key_val(v1, v1, descending=True)
                cand[pl.ds(jnp.minimum(o, CAP - 2 * L), L)] = k0
                cand[pl.ds(jnp.minimum(o + c0, CAP - L), L)] = k1
                offref[...] = offv + _splat(c0 + c1)

            return 0

        lax.fori_loop(0, NBLK, block, 0)
        tot = offref[...][0]
        cand[pl.ds(jnp.minimum(tot, CAP - L), L)] = zeros16f

        def count_ge(ref, nv, key_fn, mid):
            midv = _splat(mid)

            def cnt(j, acc):
                ci = key_fn(ref[pl.ds(j * L, L)])
                return acc + (ci >= midv).astype(jnp.int32)

            return jnp.sum(lax.fori_loop(0, nv, cnt, zeros16i))

        def bisect(ref, nv, key_fn, lo0, hi0):
            def cond(c):
                lo, hi = c
                return (hi - lo) > 1

            def body(c):
                lo, hi = c
                mid = lo + ((hi - lo) >> 1)
                ok = count_ge(ref, nv, key_fn, mid) >= K
                return (jnp.where(ok, mid, lo), jnp.where(ok, hi, mid))

            return lax.while_loop(cond, body, (lo0, hi0))[0]

        def dv_sums(ref, nv, thr):
            thrv = _splat(thr)

            def acc(j, c):
                a, aa = c
                dv = jnp.maximum(ref[pl.ds(j * L, L)] - thrv, 0.0)
                return a + dv, aa + dv * dv

            a, aa = lax.fori_loop(0, nv, acc, (zeros16f, zeros16f))
            return jnp.sum(a), jnp.sum(aa)

        def fast_path(_):
            nv = (jnp.minimum(tot, CAP - L) + (L - 1)) // L
            kf = lambda v: plsc.bitcast(v, jnp.int32)
            lob = bisect(cand, nv, kf, jnp.int32(T0_BITS), jnp.int32(HI_BITS))
            thr = jnp.max(plsc.bitcast(_splat(lob), jnp.float32))
            sv, ssv = dv_sums(cand, nv, thr)
            return sv, ssv, thr

        def slow_path(_):
            def kf(v):
                ci = plsc.bitcast(v, jnp.int32)
                ku = plsc.bitcast(
                    jnp.where(ci < 0, ~ci, ci), jnp.uint32)
                return ku ^ jnp.uint32(0x80000000)

            lob = bisect(buf, NVREG, kf, jnp.uint32(0),
                         jnp.uint32(0xFFFFFFFF))
            ks = plsc.bitcast(_splat(lob ^ jnp.uint32(0x80000000)),
                              jnp.int32)
            thrv = plsc.bitcast(jnp.where(ks < 0, ~ks, ks), jnp.float32)
            thr = jnp.max(thrv)
            sv, ssv = dv_sums(buf, NVREG, thr)
            return sv, ssv, thr

        ok_fast = jnp.logical_and(tot >= K, tot <= CAP - L)
        sv, ssv, thr = lax.cond(ok_fast, fast_path, slow_path, 0)

        lane = r_local & (L - 1)
        sel = lanes == _splat(lane)
        sacc = jnp.where(sel, _splat(sv), sacc)
        ssacc = jnp.where(sel, _splat(ssv), ssacc)
        tacc = jnp.where(sel, _splat(thr), tacc)

        @pl.when(lane == L - 1)
        def _flush():
            base = r_local - (L - 1)
            s_loc[pl.ds(base, L)] = sacc
            ss_loc[pl.ds(base, L)] = ssacc
            t_loc[pl.ds(base, L)] = tacc

        return sacc, ssacc, tacc

    zf = jnp.zeros((L,), jnp.float32)
    lax.fori_loop(0, RPW, row_body, (zf, zf, zf))
    pltpu.sync_copy(s_loc, s_hbm.at[pl.ds(row0, RPW)])
    pltpu.sync_copy(ss_loc, ss_hbm.at[pl.ds(row0, RPW)])
    pltpu.sync_copy(t_loc, t_hbm.at[pl.ds(row0, RPW)])


_sc_stats = functools.partial(
    pl.kernel,
    mesh=plsc.VectorSubcoreMesh(core_axis_name="c", subcore_axis_name="s"),
    compiler_params=pltpu.CompilerParams(needs_layout_passes=False),
    out_type=[jax.ShapeDtypeStruct((ROWS,), jnp.float32)] * 3,
    scratch_types=[
        pltpu.VMEM((V,), jnp.float32),
        pltpu.VMEM((CAP,), jnp.float32),
        pltpu.VMEM((L,), jnp.int32),
        pltpu.VMEM((RPW,), jnp.float32),
        pltpu.VMEM((RPW,), jnp.float32),
        pltpu.VMEM((RPW,), jnp.float32),
    ],
)(_sc_stats_body)


def _loss_body(xp_ref, xa_ref, sp_ref, ssp_ref, tp_ref, sa_ref, ssa_ref,
               ta_ref, scal_ref, out_ref):
    w = scal_ref[0, 0]
    b = scal_ref[0, 1]
    w2 = scal_ref[0, 2]
    w3 = scal_ref[0, 3]
    b2 = scal_ref[0, 4]
    b3 = scal_ref[0, 5]

    xp = xp_ref[...]
    xa = xa_ref[...]
    eps = jnp.float32(1e-8)
    n_p = jnp.maximum(jnp.sqrt(jnp.sum(xp * xp, axis=1, keepdims=True)), eps)
    n_a = jnp.maximum(jnp.sqrt(jnp.sum(xa * xa, axis=1, keepdims=True)), eps)
    dot = lax.dot_general(xp, xa, (((1,), (1,)), ((), ())),
                          preferred_element_type=jnp.float32)
    out_dot = dot / (n_p * n_a.T)

    kf = jnp.float32(K)
    km1 = jnp.float32(K - 1)
    mean_a = ta_ref[...] + sa_ref[...] / kf
    var_a = jnp.maximum(ssa_ref[...] - sa_ref[...] * sa_ref[...] / kf, 0.0) / km1
    std_a = jnp.sqrt(var_a)
    mean_p = tp_ref[...] + sp_ref[...] / kf
    var_p = jnp.maximum(ssp_ref[...] - sp_ref[...] * sp_ref[...] / kf, 0.0) / km1
    std_p = jnp.sqrt(var_p)

    d1 = _hs(mean_a * w2 + w3).T
    s1 = _hs(std_a * b2 + b3).T
    d2 = _hs(mean_p * w2 + w3)
    s2 = _hs(std_p * b2 + b3)

    odn = 0.5 * ((out_dot - d1) / s1 + (out_dot - d2) / s2)
    cs = odn * w + b

    rmax = jnp.max(cs, axis=1, keepdims=True)
    lse = jnp.log(jnp.sum(jnp.exp(cs - rmax), axis=1, keepdims=True)) + rmax
    ii = lax.broadcasted_iota(jnp.int32, (B, B), 0)
    jj = lax.broadcasted_iota(jnp.int32, (B, B), 1)
    diag = jnp.sum(jnp.where(ii == jj, cs, 0.0), axis=1, keepdims=True)
    out_ref[0, 0] = jnp.mean(lse - diag)


@jax.jit
def kernel(x, cosine, label, w, b, w2, w3, b2, b3):
    del label
    rows = cosine.reshape(ROWS, V)

    s, ss, t = _sc_stats(rows)

    s = s.reshape(ROWS, 1)
    ss = ss.reshape(ROWS, 1)
    t = t.reshape(ROWS, 1)
    sp, sa = s[0::2], s[1::2]
    ssp, ssa = ss[0::2], ss[1::2]
    tp, ta = t[0::2], t[1::2]
    xp = x[:, 0, :]
    xa = x[:, 1, :]
    scal = jnp.stack([w, b, w2, w3, b2, b3]).reshape(1, 6).astype(jnp.float32)

    out = pl.pallas_call(
        _loss_body,
        in_specs=[pl.BlockSpec(memory_space=pltpu.VMEM)] * 8
        + [pl.BlockSpec(memory_space=pltpu.SMEM)],
        out_specs=pl.BlockSpec(memory_space=pltpu.SMEM),
        out_shape=jax.ShapeDtypeStruct((1, 1), jnp.float32),
    )(xp, xa, sp, ssp, tp, sa, ssa, ta, scal)
    return out[0, 0]


# speedup vs baseline: 1.7338x; 1.7338x over previous
import functools

import jax
import jax.numpy as jnp
import numpy as np
from jax import lax
from jax.experimental import pallas as pl
from jax.experimental.pallas import tpu as pltpu
from jax.experimental.pallas import tpu_sc as plsc

B = 1024
D = 128
V = 100000
K = 101
ROWS = 2 * B

NC, NS, L = 2, 16, 16
NW = NC * NS
SC_ROWS = 768
TC_ROWS = ROWS - SC_ROWS
RPW = SC_ROWS // NW
LOCSZ = 32
NVREG = V // L
NBLK = NVREG // 2
CAP = 2048
T0 = 0.9975
T0_BITS = int(np.float32(T0).view(np.int32))
HI_BITS = 0x7F800000
R_BLK = 16


def _hs(t):
    return jnp.clip((t + 3.0) / 6.0, 0.0, 1.0)


def _splat(x, dtype=None):
    v = jnp.broadcast_to(x, (L,))
    return v.astype(dtype) if dtype is not None else v


def _sc_stats_body(cos_hbm, s_hbm, ss_hbm, t_hbm, buf, cand, offref, s_loc,
                   ss_loc, t_loc):
    wid = lax.axis_index("s") * NC + lax.axis_index("c")
    row0 = wid * RPW
    zeros16i = jnp.zeros((L,), jnp.int32)
    zeros16f = jnp.zeros((L,), jnp.float32)
    t0v = jnp.full((L,), T0, jnp.float32)
    lanes = lax.broadcasted_iota(jnp.int32, (L,), 0)

    def row_body(r_local, carry):
        sacc, ssacc, tacc = carry
        row = row0 + r_local
        pltpu.sync_copy(cos_hbm.at[row], buf)
        offref[...] = zeros16i

        def block(i, unused):
            v0 = buf[pl.ds(i * (2 * L), L)]
            v1 = buf[pl.ds(i * (2 * L) + L, L)]
            hit = jnp.max(jnp.maximum(v0, v1)) > T0

            @pl.when(hit)
            def _commit():
                offv = offref[...]
                o = offv[0]
                c0 = jnp.sum((v0 > t0v).astype(jnp.int32))
                c1 = jnp.sum((v1 > t0v).astype(jnp.int32))
                k0, _ = plsc.sort_key_val(v0, v0, descending=True)
                k1, _ = plsc.sort_key_val(v1, v1, descending=True)
                cand[pl.ds(jnp.minimum(o, CAP - 2 * L), L)] = k0
                cand[pl.ds(jnp.minimum(o + c0, CAP - L), L)] = k1
                offref[...] = offv + _splat(c0 + c1)

            return 0

        lax.fori_loop(0, NBLK, block, 0)
        tot = offref[...][0]
        cand[pl.ds(jnp.minimum(tot, CAP - L), L)] = zeros16f

        def count_ge(ref, nv, key_fn, mid):
            midv = _splat(mid)

            def cnt(j, acc):
                ci = key_fn(ref[pl.ds(j * L, L)])
                return acc + (ci >= midv).astype(jnp.int32)

            return jnp.sum(lax.fori_loop(0, nv, cnt, zeros16i))

        def bisect(ref, nv, key_fn, lo0, hi0):
            def cond(c):
                lo, hi = c
                return (hi - lo) > 1

            def body(c):
                lo, hi = c
                mid = lo + ((hi - lo) >> 1)
                ok = count_ge(ref, nv, key_fn, mid) >= K
                return (jnp.where(ok, mid, lo), jnp.where(ok, hi, mid))

            return lax.while_loop(cond, body, (lo0, hi0))[0]

        def dv_sums(ref, nv, thr):
            thrv = _splat(thr)

            def acc(j, c):
                a, aa = c
                dv = jnp.maximum(ref[pl.ds(j * L, L)] - thrv, 0.0)
                return a + dv, aa + dv * dv

            a, aa = lax.fori_loop(0, nv, acc, (zeros16f, zeros16f))
            return jnp.sum(a), jnp.sum(aa)

        def fast_path(_):
            nv = (jnp.minimum(tot, CAP - L) + (L - 1)) // L
            kf = lambda v: plsc.bitcast(v, jnp.int32)
            lob = bisect(cand, nv, kf, jnp.int32(T0_BITS), jnp.int32(HI_BITS))
            thr = jnp.max(plsc.bitcast(_splat(lob), jnp.float32))
            sv, ssv = dv_sums(cand, nv, thr)
            return sv, ssv, thr

        def slow_path(_):
            def kf(v):
                ci = plsc.bitcast(v, jnp.int32)
                ku = plsc.bitcast(
                    jnp.where(ci < 0, ~ci, ci), jnp.uint32)
                return ku ^ jnp.uint32(0x80000000)

            lob = bisect(buf, NVREG, kf, jnp.uint32(0),
                         jnp.uint32(0xFFFFFFFF))
            ks = plsc.bitcast(_splat(lob ^ jnp.uint32(0x80000000)),
                              jnp.int32)
            thrv = plsc.bitcast(jnp.where(ks < 0, ~ks, ks), jnp.float32)
            thr = jnp.max(thrv)
            sv, ssv = dv_sums(buf, NVREG, thr)
            return sv, ssv, thr

        ok_fast = jnp.logical_and(tot >= K, tot <= CAP - L)
        sv, ssv, thr = lax.cond(ok_fast, fast_path, slow_path, 0)

        lane = r_local & (L - 1)
        sel = lanes == _splat(lane)
        sacc = jnp.where(sel, _splat(sv), sacc)
        ssacc = jnp.where(sel, _splat(ssv), ssacc)
        tacc = jnp.where(sel, _splat(thr), tacc)

        @pl.when(lane == L - 1)
        def _flush():
            base = r_local - (L - 1)
            s_loc[pl.ds(base, L)] = sacc
            ss_loc[pl.ds(base, L)] = ssacc
            t_loc[pl.ds(base, L)] = tacc

        return sacc, ssacc, tacc

    zf = jnp.zeros((L,), jnp.float32)
    fin_s, fin_ss, fin_t = lax.fori_loop(0, RPW, row_body, (zf, zf, zf))
    if RPW % L:
        base = RPW - (RPW % L)
        s_loc[pl.ds(base, L)] = fin_s
        ss_loc[pl.ds(base, L)] = fin_ss
        t_loc[pl.ds(base, L)] = fin_t
    pltpu.sync_copy(s_loc.at[pl.ds(0, RPW)], s_hbm.at[pl.ds(row0, RPW)])
    pltpu.sync_copy(ss_loc.at[pl.ds(0, RPW)], ss_hbm.at[pl.ds(row0, RPW)])
    pltpu.sync_copy(t_loc.at[pl.ds(0, RPW)], t_hbm.at[pl.ds(row0, RPW)])


_sc_stats = functools.partial(
    pl.kernel,
    mesh=plsc.VectorSubcoreMesh(core_axis_name="c", subcore_axis_name="s"),
    compiler_params=pltpu.CompilerParams(needs_layout_passes=False),
    out_type=[jax.ShapeDtypeStruct((SC_ROWS,), jnp.float32)] * 3,
    scratch_types=[
        pltpu.VMEM((V,), jnp.float32),
        pltpu.VMEM((CAP,), jnp.float32),
        pltpu.VMEM((L,), jnp.int32),
        pltpu.VMEM((LOCSZ,), jnp.float32),
        pltpu.VMEM((LOCSZ,), jnp.float32),
        pltpu.VMEM((LOCSZ,), jnp.float32),
    ],
)(_sc_stats_body)


def _tc_stats_body(cos_ref, s_ref, ss_ref, t_ref):
    v = cos_ref[...]
    vi = lax.bitcast_convert_type(v, jnp.int32)

    lo = jnp.zeros((R_BLK, 1), jnp.int32)
    hi = jnp.full((R_BLK, 1), 0x40000000, jnp.int32)

    def step(_, carry):
        lo, hi = carry
        mid = (lo + hi) >> 1
        cnt = jnp.sum((vi >= mid).astype(jnp.int32), axis=1, keepdims=True)
        pred = cnt >= K
        return jnp.where(pred, mid, lo), jnp.where(pred, hi, mid)

    lo, hi = lax.fori_loop(0, 31, step, (lo, hi))
    thr = lax.bitcast_convert_type(lo, jnp.float32)

    dv = jnp.maximum(v - thr, 0.0)
    s_ref[...] = jnp.sum(dv, axis=1, keepdims=True)
    ss_ref[...] = jnp.sum(dv * dv, axis=1, keepdims=True)
    t_ref[...] = thr


def _loss_body(xp_ref, xa_ref, sp_ref, ssp_ref, tp_ref, sa_ref, ssa_ref,
               ta_ref, scal_ref, out_ref):
    w = scal_ref[0, 0]
    b = scal_ref[0, 1]
    w2 = scal_ref[0, 2]
    w3 = scal_ref[0, 3]
    b2 = scal_ref[0, 4]
    b3 = scal_ref[0, 5]

    xp = xp_ref[...]
    xa = xa_ref[...]
    eps = jnp.float32(1e-8)
    n_p = jnp.maximum(jnp.sqrt(jnp.sum(xp * xp, axis=1, keepdims=True)), eps)
    n_a = jnp.maximum(jnp.sqrt(jnp.sum(xa * xa, axis=1, keepdims=True)), eps)
    dot = lax.dot_general(xp, xa, (((1,), (1,)), ((), ())),
                          preferred_element_type=jnp.float32)
    out_dot = dot / (n_p * n_a.T)

    kf = jnp.float32(K)
    km1 = jnp.float32(K - 1)
    mean_a = ta_ref[...] + sa_ref[...] / kf
    var_a = jnp.maximum(ssa_ref[...] - sa_ref[...] * sa_ref[...] / kf, 0.0) / km1
    std_a = jnp.sqrt(var_a)
    mean_p = tp_ref[...] + sp_ref[...] / kf
    var_p = jnp.maximum(ssp_ref[...] - sp_ref[...] * sp_ref[...] / kf, 0.0) / km1
    std_p = jnp.sqrt(var_p)

    d1 = _hs(mean_a * w2 + w3).T
    s1 = _hs(std_a * b2 + b3).T
    d2 = _hs(mean_p * w2 + w3)
    s2 = _hs(std_p * b2 + b3)

    odn = 0.5 * ((out_dot - d1) / s1 + (out_dot - d2) / s2)
    cs = odn * w + b

    rmax = jnp.max(cs, axis=1, keepdims=True)
    lse = jnp.log(jnp.sum(jnp.exp(cs - rmax), axis=1, keepdims=True)) + rmax
    ii = lax.broadcasted_iota(jnp.int32, (B, B), 0)
    jj = lax.broadcasted_iota(jnp.int32, (B, B), 1)
    diag = jnp.sum(jnp.where(ii == jj, cs, 0.0), axis=1, keepdims=True)
    out_ref[0, 0] = jnp.mean(lse - diag)


@jax.jit
def kernel(x, cosine, label, w, b, w2, w3, b2, b3):
    del label
    rows = cosine.reshape(ROWS, V)

    s1, ss1, t1 = _sc_stats(rows[:SC_ROWS])

    s2, ss2, t2 = pl.pallas_call(
        _tc_stats_body,
        grid=(TC_ROWS // R_BLK,),
        in_specs=[pl.BlockSpec((R_BLK, V), lambda i: (i, 0))],
        out_specs=[pl.BlockSpec((R_BLK, 1), lambda i: (i, 0))] * 3,
        out_shape=[jax.ShapeDtypeStruct((TC_ROWS, 1), jnp.float32)] * 3,
    )(rows[SC_ROWS:])

    s = jnp.concatenate([s1.reshape(SC_ROWS, 1), s2])
    ss = jnp.concatenate([ss1.reshape(SC_ROWS, 1), ss2])
    t = jnp.concatenate([t1.reshape(SC_ROWS, 1), t2])
    sp, sa = s[0::2], s[1::2]
    ssp, ssa = ss[0::2], ss[1::2]
    tp, ta = t[0::2], t[1::2]
    xp = x[:, 0, :]
    xa = x[:, 1, :]
    scal = jnp.stack([w, b, w2, w3, b2, b3]).reshape(1, 6).astype(jnp.float32)

    out = pl.pallas_call(
        _loss_body,
        in_specs=[pl.BlockSpec(memory_space=pltpu.VMEM)] * 8
        + [pl.BlockSpec(memory_space=pltpu.SMEM)],
        out_specs=pl.BlockSpec(memory_space=pltpu.SMEM),
        out_shape=jax.ShapeDtypeStruct((1, 1), jnp.float32),
    )(xp, xa, sp, ssp, tp, sa, ssa, ta, scal)
    return out[0, 0]


# speedup vs baseline: 1.9612x; 1.1312x over previous
import functools

import jax
import jax.numpy as jnp
import numpy as np
from jax import lax
from jax.experimental import pallas as pl
from jax.experimental.pallas import tpu as pltpu
from jax.experimental.pallas import tpu_sc as plsc

B = 1024
D = 128
V = 100000
K = 101
ROWS = 2 * B

NC, NS, L = 2, 16, 16
NW = NC * NS
SC_ROWS = 768
TC_ROWS = ROWS - SC_ROWS
RPW = SC_ROWS // NW
LOCSZ = 32
NVREG = V // L
NBLK = NVREG // 2
CAP = 2048
T0 = 0.9975
T0_BITS = int(np.float32(T0).view(np.int32))
HI_BITS = 0x7F800000
R_BLK = 16


def _hs(t):
    return jnp.clip((t + 3.0) / 6.0, 0.0, 1.0)


def _splat(x, dtype=None):
    v = jnp.broadcast_to(x, (L,))
    return v.astype(dtype) if dtype is not None else v


def _sc_stats_body(cos_hbm, s_hbm, ss_hbm, t_hbm, buf, cand, offref, s_loc,
                   ss_loc, t_loc):
    wid = lax.axis_index("s") * NC + lax.axis_index("c")
    row0 = wid * RPW
    zeros16i = jnp.zeros((L,), jnp.int32)
    zeros16f = jnp.zeros((L,), jnp.float32)
    t0v = jnp.full((L,), T0, jnp.float32)
    lanes = lax.broadcasted_iota(jnp.int32, (L,), 0)

    def row_body(r_local, carry):
        sacc, ssacc, tacc = carry
        row = row0 + r_local
        pltpu.sync_copy(cos_hbm.at[row], buf)
        offref[...] = zeros16i

        def block(i, unused):
            v0 = buf[pl.ds(i * (2 * L), L)]
            v1 = buf[pl.ds(i * (2 * L) + L, L)]
            hit = jnp.max(jnp.maximum(v0, v1)) > T0

            @pl.when(hit)
            def _commit():
                offv = offref[...]
                o = offv[0]
                c0 = jnp.sum((v0 > t0v).astype(jnp.int32))
                c1 = jnp.sum((v1 > t0v).astype(jnp.int32))
                k0, _ = plsc.sort_key_val(v0, v0, descending=True)
                k1, _ = plsc.sort_key_val(v1, v1, descending=True)
                cand[pl.ds(jnp.minimum(o, CAP - 2 * L), L)] = k0
                cand[pl.ds(jnp.minimum(o + c0, CAP - L), L)] = k1
                offref[...] = offv + _splat(c0 + c1)

            return 0

        lax.fori_loop(0, NBLK, block, 0)
        tot = offref[...][0]
        cand[pl.ds(jnp.minimum(tot, CAP - L), L)] = zeros16f

        def count_ge(ref, nv, key_fn, mid):
            midv = _splat(mid)

            def cnt(j, acc):
                ci = key_fn(ref[pl.ds(j * L, L)])
                return acc + (ci >= midv).astype(jnp.int32)

            return jnp.sum(lax.fori_loop(0, nv, cnt, zeros16i))

        def bisect(ref, nv, key_fn, lo0, hi0):
            def cond(c):
                lo, hi = c
                return (hi - lo) > 1

            def body(c):
                lo, hi = c
                mid = lo + ((hi - lo) >> 1)
                ok = count_ge(ref, nv, key_fn, mid) >= K
                return (jnp.where(ok, mid, lo), jnp.where(ok, hi, mid))

            return lax.while_loop(cond, body, (lo0, hi0))[0]

        def dv_sums(ref, nv, thr):
            thrv = _splat(thr)

            def acc(j, c):
                a, aa = c
                dv = jnp.maximum(ref[pl.ds(j * L, L)] - thrv, 0.0)
                return a + dv, aa + dv * dv

            a, aa = lax.fori_loop(0, nv, acc, (zeros16f, zeros16f))
            return jnp.sum(a), jnp.sum(aa)

        def fast_path(_):
            nv = (jnp.minimum(tot, CAP - L) + (L - 1)) // L
            kf = lambda v: plsc.bitcast(v, jnp.int32)
            lob = bisect(cand, nv, kf, jnp.int32(T0_BITS), jnp.int32(HI_BITS))
            thr = jnp.max(plsc.bitcast(_splat(lob), jnp.float32))
            sv, ssv = dv_sums(cand, nv, thr)
            return sv, ssv, thr

        def slow_path(_):
            def kf(v):
                ci = plsc.bitcast(v, jnp.int32)
                ku = plsc.bitcast(
                    jnp.where(ci < 0, ~ci, ci), jnp.uint32)
                return ku ^ jnp.uint32(0x80000000)

            lob = bisect(buf, NVREG, kf, jnp.uint32(0),
                         jnp.uint32(0xFFFFFFFF))
            ks = plsc.bitcast(_splat(lob ^ jnp.uint32(0x80000000)),
                              jnp.int32)
            thrv = plsc.bitcast(jnp.where(ks < 0, ~ks, ks), jnp.float32)
            thr = jnp.max(thrv)
            sv, ssv = dv_sums(buf, NVREG, thr)
            return sv, ssv, thr

        ok_fast = jnp.logical_and(tot >= K, tot <= CAP - L)
        sv, ssv, thr = lax.cond(ok_fast, fast_path, slow_path, 0)

        lane = r_local & (L - 1)
        sel = lanes == _splat(lane)
        sacc = jnp.where(sel, _splat(sv), sacc)
        ssacc = jnp.where(sel, _splat(ssv), ssacc)
        tacc = jnp.where(sel, _splat(thr), tacc)

        @pl.when(lane == L - 1)
        def _flush():
            base = r_local - (L - 1)
            s_loc[pl.ds(base, L)] = sacc
            ss_loc[pl.ds(base, L)] = ssacc
            t_loc[pl.ds(base, L)] = tacc

        return sacc, ssacc, tacc

    zf = jnp.zeros((L,), jnp.float32)
    fin_s, fin_ss, fin_t = lax.fori_loop(0, RPW, row_body, (zf, zf, zf))
    if RPW % L:
        base = RPW - (RPW % L)
        s_loc[pl.ds(base, L)] = fin_s
        ss_loc[pl.ds(base, L)] = fin_ss
        t_loc[pl.ds(base, L)] = fin_t
    pltpu.sync_copy(s_loc.at[pl.ds(0, RPW)], s_hbm.at[pl.ds(row0, RPW)])
    pltpu.sync_copy(ss_loc.at[pl.ds(0, RPW)], ss_hbm.at[pl.ds(row0, RPW)])
    pltpu.sync_copy(t_loc.at[pl.ds(0, RPW)], t_hbm.at[pl.ds(row0, RPW)])


_sc_stats = functools.partial(
    pl.kernel,
    mesh=plsc.VectorSubcoreMesh(core_axis_name="c", subcore_axis_name="s"),
    compiler_params=pltpu.CompilerParams(needs_layout_passes=False),
    out_type=[jax.ShapeDtypeStruct((SC_ROWS,), jnp.float32)] * 3,
    scratch_types=[
        pltpu.VMEM((V,), jnp.float32),
        pltpu.VMEM((CAP,), jnp.float32),
        pltpu.VMEM((L,), jnp.int32),
        pltpu.VMEM((LOCSZ,), jnp.float32),
        pltpu.VMEM((LOCSZ,), jnp.float32),
        pltpu.VMEM((LOCSZ,), jnp.float32),
    ],
)(_sc_stats_body)


def _tc_stats_body(cos_ref, s_ref, ss_ref, t_ref):
    v = cos_ref[...]
    vi = lax.bitcast_convert_type(v, jnp.int32)

    lo = jnp.zeros((R_BLK, 1), jnp.int32)
    hi = jnp.full((R_BLK, 1), 0x40000000, jnp.int32)

    def step(_, carry):
        lo, hi = carry
        mid = (lo + hi) >> 1
        cnt = jnp.sum((vi >= mid).astype(jnp.int32), axis=1, keepdims=True)
        pred = cnt >= K
        return jnp.where(pred, mid, lo), jnp.where(pred, hi, mid)

    lo, hi = lax.fori_loop(0, 31, step, (lo, hi))
    thr = lax.bitcast_convert_type(lo, jnp.float32)

    dv = jnp.maximum(v - thr, 0.0)
    s_ref[...] = jnp.sum(dv, axis=1, keepdims=True)
    ss_ref[...] = jnp.sum(dv * dv, axis=1, keepdims=True)
    t_ref[...] = thr


def _loss_body(xp_ref, xa_ref, sp_ref, ssp_ref, tp_ref, sa_ref, ssa_ref,
               ta_ref, scal_ref, out_ref):
    w = scal_ref[0, 0]
    b = scal_ref[0, 1]
    w2 = scal_ref[0, 2]
    w3 = scal_ref[0, 3]
    b2 = scal_ref[0, 4]
    b3 = scal_ref[0, 5]

    xp = xp_ref[...]
    xa = xa_ref[...]
    eps = jnp.float32(1e-8)
    n_p = jnp.maximum(jnp.sqrt(jnp.sum(xp * xp, axis=1, keepdims=True)), eps)
    n_a = jnp.maximum(jnp.sqrt(jnp.sum(xa * xa, axis=1, keepdims=True)), eps)
    dot = lax.dot_general(xp, xa, (((1,), (1,)), ((), ())),
                          preferred_element_type=jnp.float32)
    out_dot = dot / (n_p * n_a.T)

    kf = jnp.float32(K)
    km1 = jnp.float32(K - 1)
    mean_a = ta_ref[...] + sa_ref[...] / kf
    var_a = jnp.maximum(ssa_ref[...] - sa_ref[...] * sa_ref[...] / kf, 0.0) / km1
    std_a = jnp.sqrt(var_a)
    mean_p = tp_ref[...] + sp_ref[...] / kf
    var_p = jnp.maximum(ssp_ref[...] - sp_ref[...] * sp_ref[...] / kf, 0.0) / km1
    std_p = jnp.sqrt(var_p)

    d1 = _hs(mean_a * w2 + w3).T
    s1 = _hs(std_a * b2 + b3).T
    d2 = _hs(mean_p * w2 + w3)
    s2 = _hs(std_p * b2 + b3)

    odn = 0.5 * ((out_dot - d1) / s1 + (out_dot - d2) / s2)
    cs = odn * w + b

    rmax = jnp.max(cs, axis=1, keepdims=True)
    lse = jnp.log(jnp.sum(jnp.exp(cs - rmax), axis=1, keepdims=True)) + rmax
    ii = lax.broadcasted_iota(jnp.int32, (B, B), 0)
    jj = lax.broadcasted_iota(jnp.int32, (B, B), 1)
    diag = jnp.sum(jnp.where(ii == jj, cs, 0.0), axis=1, keepdims=True)
    out_ref[0, 0] = jnp.mean(lse - diag)


@jax.jit
def kernel(x, cosine, label, w, b, w2, w3, b2, b3):
    del label
    rows = cosine.reshape(ROWS, V)

    s1, ss1, t1 = _sc_stats(rows)

    s2, ss2, t2 = pl.pallas_call(
        _tc_stats_body,
        grid=(TC_ROWS // R_BLK,),
        in_specs=[pl.BlockSpec((R_BLK, V),
                               lambda i: (i + SC_ROWS // R_BLK, 0))],
        out_specs=[pl.BlockSpec((R_BLK, 1), lambda i: (i, 0))] * 3,
        out_shape=[jax.ShapeDtypeStruct((TC_ROWS, 1), jnp.float32)] * 3,
    )(rows)

    s = jnp.concatenate([s1.reshape(SC_ROWS, 1), s2])
    ss = jnp.concatenate([ss1.reshape(SC_ROWS, 1), ss2])
    t = jnp.concatenate([t1.reshape(SC_ROWS, 1), t2])
    sp, sa = s[0::2], s[1::2]
    ssp, ssa = ss[0::2], ss[1::2]
    tp, ta = t[0::2], t[1::2]
    xp = x[:, 0, :]
    xa = x[:, 1, :]
    scal = jnp.stack([w, b, w2, w3, b2, b3]).reshape(1, 6).astype(jnp.float32)

    out = pl.pallas_call(
        _loss_body,
        in_specs=[pl.BlockSpec(memory_space=pltpu.VMEM)] * 8
        + [pl.BlockSpec(memory_space=pltpu.SMEM)],
        out_specs=pl.BlockSpec(memory_space=pltpu.SMEM),
        out_shape=jax.ShapeDtypeStruct((1, 1), jnp.float32),
    )(xp, xa, sp, ssp, tp, sa, ssa, ta, scal)
    return out[0, 0]


# speedup vs baseline: 1.9627x; 1.0008x over previous
import functools

import jax
import jax.numpy as jnp
import numpy as np
from jax import lax
from jax.experimental import pallas as pl
from jax.experimental.pallas import tpu as pltpu
from jax.experimental.pallas import tpu_sc as plsc

B = 1024
D = 128
V = 100000
K = 101
ROWS = 2 * B

NC, NS, L = 2, 16, 16
NW = NC * NS
SC_ROWS = 768
TC_ROWS = ROWS - SC_ROWS
RPW = SC_ROWS // NW
LOCSZ = 32
NVREG = V // L
NBLK = NVREG // 2
CAP = 2048
T0 = 0.9975
T0_BITS = int(np.float32(T0).view(np.int32))
HI_BITS = 0x7F800000
R_BLK = 16


def _hs(t):
    return jnp.clip((t + 3.0) / 6.0, 0.0, 1.0)


def _splat(x, dtype=None):
    v = jnp.broadcast_to(x, (L,))
    return v.astype(dtype) if dtype is not None else v


def _sc_stats_body(cos_hbm, s_hbm, ss_hbm, t_hbm, buf, cand, offref, s_loc,
                   ss_loc, t_loc):
    wid = lax.axis_index("s") * NC + lax.axis_index("c")
    row0 = wid * RPW
    zeros16i = jnp.zeros((L,), jnp.int32)
    zeros16f = jnp.zeros((L,), jnp.float32)
    t0v = jnp.full((L,), T0, jnp.float32)
    lanes = lax.broadcasted_iota(jnp.int32, (L,), 0)

    def row_body(r_local, carry):
        sacc, ssacc, tacc = carry
        row = row0 + r_local
        pltpu.sync_copy(cos_hbm.at[row], buf)
        offref[...] = zeros16i

        def block(i, unused):
            v0 = buf[pl.ds(i * (2 * L), L)]
            v1 = buf[pl.ds(i * (2 * L) + L, L)]
            hit = jnp.max(jnp.maximum(v0, v1)) > T0

            @pl.when(hit)
            def _commit():
                offv = offref[...]
                o = offv[0]
                c0 = jnp.sum((v0 > t0v).astype(jnp.int32))
                c1 = jnp.sum((v1 > t0v).astype(jnp.int32))
                k0, _ = plsc.sort_key_val(v0, v0, descending=True)
                k1, _ = plsc.sort_key_val(v1, v1, descending=True)
                cand[pl.ds(jnp.minimum(o, CAP - 2 * L), L)] = k0
                cand[pl.ds(jnp.minimum(o + c0, CAP - L), L)] = k1
                offref[...] = offv + _splat(c0 + c1)

            return 0

        lax.fori_loop(0, NBLK, block, 0)
        tot = offref[...][0]
        cand[pl.ds(jnp.minimum(tot, CAP - L), L)] = zeros16f

        def count_ge(ref, nv, key_fn, mid):
            midv = _splat(mid)

            def cnt(j, acc):
                ci = key_fn(ref[pl.ds(j * L, L)])
                return acc + (ci >= midv).astype(jnp.int32)

            return jnp.sum(lax.fori_loop(0, nv, cnt, zeros16i))

        def bisect(ref, nv, key_fn, lo0, hi0):
            def cond(c):
                lo, hi = c
                return (hi - lo) > 1

            def body(c):
                lo, hi = c
                mid = lo + ((hi - lo) >> 1)
                ok = count_ge(ref, nv, key_fn, mid) >= K
                return (jnp.where(ok, mid, lo), jnp.where(ok, hi, mid))

            return lax.while_loop(cond, body, (lo0, hi0))[0]

        def dv_sums(ref, nv, thr):
            thrv = _splat(thr)

            def acc(j, c):
                a, aa = c
                dv = jnp.maximum(ref[pl.ds(j * L, L)] - thrv, 0.0)
                return a + dv, aa + dv * dv

            a, aa = lax.fori_loop(0, nv, acc, (zeros16f, zeros16f))
            return jnp.sum(a), jnp.sum(aa)

        def fast_path(_):
            nv = (jnp.minimum(tot, CAP - L) + (L - 1)) // L
            kf = lambda v: plsc.bitcast(v, jnp.int32)
            lob = bisect(cand, nv, kf, jnp.int32(T0_BITS), jnp.int32(HI_BITS))
            thr = jnp.max(plsc.bitcast(_splat(lob), jnp.float32))
            sv, ssv = dv_sums(cand, nv, thr)
            return sv, ssv, thr

        def slow_path(_):
            def kf(v):
                ci = plsc.bitcast(v, jnp.int32)
                ku = plsc.bitcast(
                    jnp.where(ci < 0, ~ci, ci), jnp.uint32)
                return ku ^ jnp.uint32(0x80000000)

            lob = bisect(buf, NVREG, kf, jnp.uint32(0),
                         jnp.uint32(0xFFFFFFFF))
            ks = plsc.bitcast(_splat(lob ^ jnp.uint32(0x80000000)),
                              jnp.int32)
            thrv = plsc.bitcast(jnp.where(ks < 0, ~ks, ks), jnp.float32)
            thr = jnp.max(thrv)
            sv, ssv = dv_sums(buf, NVREG, thr)
            return sv, ssv, thr

        ok_fast = jnp.logical_and(tot >= K, tot <= CAP - L)
        sv, ssv, thr = lax.cond(ok_fast, fast_path, slow_path, 0)

        lane = r_local & (L - 1)
        sel = lanes == _splat(lane)
        sacc = jnp.where(sel, _splat(sv), sacc)
        ssacc = jnp.where(sel, _splat(ssv), ssacc)
        tacc = jnp.where(sel, _splat(thr), tacc)

        @pl.when(lane == L - 1)
        def _flush():
            base = r_local - (L - 1)
            s_loc[pl.ds(base, L)] = sacc
            ss_loc[pl.ds(base, L)] = ssacc
            t_loc[pl.ds(base, L)] = tacc

        return sacc, ssacc, tacc

    zf = jnp.zeros((L,), jnp.float32)
    fin_s, fin_ss, fin_t = lax.fori_loop(0, RPW, row_body, (zf, zf, zf))
    if RPW % L:
        base = RPW - (RPW % L)
        s_loc[pl.ds(base, L)] = fin_s
        ss_loc[pl.ds(base, L)] = fin_ss
        t_loc[pl.ds(base, L)] = fin_t
    pltpu.sync_copy(s_loc.at[pl.ds(0, RPW)], s_hbm.at[pl.ds(row0, RPW)])
    pltpu.sync_copy(ss_loc.at[pl.ds(0, RPW)], ss_hbm.at[pl.ds(row0, RPW)])
    pltpu.sync_copy(t_loc.at[pl.ds(0, RPW)], t_hbm.at[pl.ds(row0, RPW)])


_sc_stats = functools.partial(
    pl.kernel,
    mesh=plsc.VectorSubcoreMesh(core_axis_name="c", subcore_axis_name="s"),
    compiler_params=pltpu.CompilerParams(needs_layout_passes=False),
    out_type=[jax.ShapeDtypeStruct((SC_ROWS,), jnp.float32)] * 3,
    scratch_types=[
        pltpu.VMEM((V,), jnp.float32),
        pltpu.VMEM((CAP,), jnp.float32),
        pltpu.VMEM((L,), jnp.int32),
        pltpu.VMEM((LOCSZ,), jnp.float32),
        pltpu.VMEM((LOCSZ,), jnp.float32),
        pltpu.VMEM((LOCSZ,), jnp.float32),
    ],
)(_sc_stats_body)


def _tc_stats_body(cos_ref, s_ref, ss_ref, t_ref):
    v = cos_ref[...]
    vi = lax.bitcast_convert_type(v, jnp.int32)

    cnt0 = jnp.sum((vi >= T0_BITS).astype(jnp.int32), axis=1, keepdims=True)
    fastr = cnt0 >= K
    lo = jnp.where(fastr, jnp.int32(T0_BITS), jnp.int32(0))
    hi = jnp.where(fastr, jnp.int32(0x3F800001), jnp.int32(T0_BITS))

    def wcond(carry):
        lo, hi = carry
        return jnp.any(hi - lo > 1)

    def step(carry):
        lo, hi = carry
        mid = (lo + hi) >> 1
        cnt = jnp.sum((vi >= mid).astype(jnp.int32), axis=1, keepdims=True)
        pred = cnt >= K
        return jnp.where(pred, mid, lo), jnp.where(pred, hi, mid)

    lo, hi = lax.while_loop(wcond, step, (lo, hi))
    thr = lax.bitcast_convert_type(lo, jnp.float32)

    dv = jnp.maximum(v - thr, 0.0)
    s_ref[...] = jnp.sum(dv, axis=1, keepdims=True)
    ss_ref[...] = jnp.sum(dv * dv, axis=1, keepdims=True)
    t_ref[...] = thr


def _loss_body(xp_ref, xa_ref, sp_ref, ssp_ref, tp_ref, sa_ref, ssa_ref,
               ta_ref, scal_ref, out_ref):
    w = scal_ref[0, 0]
    b = scal_ref[0, 1]
    w2 = scal_ref[0, 2]
    w3 = scal_ref[0, 3]
    b2 = scal_ref[0, 4]
    b3 = scal_ref[0, 5]

    xp = xp_ref[...]
    xa = xa_ref[...]
    eps = jnp.float32(1e-8)
    n_p = jnp.maximum(jnp.sqrt(jnp.sum(xp * xp, axis=1, keepdims=True)), eps)
    n_a = jnp.maximum(jnp.sqrt(jnp.sum(xa * xa, axis=1, keepdims=True)), eps)
    dot = lax.dot_general(xp, xa, (((1,), (1,)), ((), ())),
                          preferred_element_type=jnp.float32)
    out_dot = dot / (n_p * n_a.T)

    kf = jnp.float32(K)
    km1 = jnp.float32(K - 1)
    mean_a = ta_ref[...] + sa_ref[...] / kf
    var_a = jnp.maximum(ssa_ref[...] - sa_ref[...] * sa_ref[...] / kf, 0.0) / km1
    std_a = jnp.sqrt(var_a)
    mean_p = tp_ref[...] + sp_ref[...] / kf
    var_p = jnp.maximum(ssp_ref[...] - sp_ref[...] * sp_ref[...] / kf, 0.0) / km1
    std_p = jnp.sqrt(var_p)

    d1 = _hs(mean_a * w2 + w3).T
    s1 = _hs(std_a * b2 + b3).T
    d2 = _hs(mean_p * w2 + w3)
    s2 = _hs(std_p * b2 + b3)

    odn = 0.5 * ((out_dot - d1) / s1 + (out_dot - d2) / s2)
    cs = odn * w + b

    rmax = jnp.max(cs, axis=1, keepdims=True)
    lse = jnp.log(jnp.sum(jnp.exp(cs - rmax), axis=1, keepdims=True)) + rmax
    ii = lax.broadcasted_iota(jnp.int32, (B, B), 0)
    jj = lax.broadcasted_iota(jnp.int32, (B, B), 1)
    diag = jnp.sum(jnp.where(ii == jj, cs, 0.0), axis=1, keepdims=True)
    out_ref[0, 0] = jnp.mean(lse - diag)


@jax.jit
def kernel(x, cosine, label, w, b, w2, w3, b2, b3):
    del label
    rows = cosine.reshape(ROWS, V)

    s1, ss1, t1 = _sc_stats(rows)

    s2, ss2, t2 = pl.pallas_call(
        _tc_stats_body,
        grid=(TC_ROWS // R_BLK,),
        in_specs=[pl.BlockSpec((R_BLK, V),
                               lambda i: (i + SC_ROWS // R_BLK, 0))],
        out_specs=[pl.BlockSpec((R_BLK, 1), lambda i: (i, 0))] * 3,
        out_shape=[jax.ShapeDtypeStruct((TC_ROWS, 1), jnp.float32)] * 3,
    )(rows)

    s = jnp.concatenate([s1.reshape(SC_ROWS, 1), s2])
    ss = jnp.concatenate([ss1.reshape(SC_ROWS, 1), ss2])
    t = jnp.concatenate([t1.reshape(SC_ROWS, 1), t2])
    sp, sa = s[0::2], s[1::2]
    ssp, ssa = ss[0::2], ss[1::2]
    tp, ta = t[0::2], t[1::2]
    xp = x[:, 0, :]
    xa = x[:, 1, :]
    scal = jnp.stack([w, b, w2, w3, b2, b3]).reshape(1, 6).astype(jnp.float32)

    out = pl.pallas_call(
        _loss_body,
        in_specs=[pl.BlockSpec(memory_space=pltpu.VMEM)] * 8
        + [pl.BlockSpec(memory_space=pltpu.SMEM)],
        out_specs=pl.BlockSpec(memory_space=pltpu.SMEM),
        out_shape=jax.ShapeDtypeStruct((1, 1), jnp.float32),
    )(xp, xa, sp, ssp, tp, sa, ssa, ta, scal)
    return out[0, 0]
